# Initial kernel scaffold; baseline (speedup 1.0000x reference)
import jax
import jax.numpy as jnp
from jax.experimental import pallas as pl


def kernel(enc, embed):
    raise NotImplementedError("write your pallas kernel here")


# speedup vs baseline: 1.6359x; 1.6359x over previous
import functools

import jax
import jax.numpy as jnp
from jax import lax
from jax.experimental import pallas as pl
from jax.experimental.pallas import tpu as pltpu
from jax.experimental.pallas import tpu_sc as plsc

K = 512
D = 64
TB = 1024

_NC = 2
_NS = 16
_NW = _NC * _NS
_CHUNK = 128


def _tc_body(x_ref, e_ref, idx_ref, loss_ref):
    i = pl.program_id(0)
    x = x_ref[...]
    e = e_ref[...]
    s = lax.dot_general(x, e, (((1,), (0,)), ((), ())),
                        preferred_element_type=jnp.float32)
    q2 = jnp.sum(x * x, axis=1, keepdims=True)
    e2 = jnp.sum(e * e, axis=0)[None, :]
    d2 = q2 + e2 - 2.0 * s
    d2 = jnp.maximum(d2, 0.0)
    idx_ref[...] = jnp.argmin(d2, axis=1).astype(jnp.int32)
    part = jnp.sum(jnp.min(d2, axis=1))

    @pl.when(i == 0)
    def _():
        loss_ref[0] = 0.0

    loss_ref[0] += part

    @pl.when(i == pl.num_programs(0) - 1)
    def _():
        loss_ref[0] = loss_ref[0] * (2.0 / (pl.num_programs(0) * TB * D))


def _tc_assign(x, embed_t):
    nt = x.shape[0]
    grid = nt // TB
    return pl.pallas_call(
        _tc_body,
        grid=(grid,),
        in_specs=[
            pl.BlockSpec((TB, D), lambda i: (i, 0)),
            pl.BlockSpec((D, K), lambda i: (0, 0)),
        ],
        out_specs=[
            pl.BlockSpec((TB,), lambda i: (i,)),
            pl.BlockSpec(memory_space=pltpu.SMEM),
        ],
        out_shape=[
            jax.ShapeDtypeStruct((nt,), jnp.int32),
            jax.ShapeDtypeStruct((1,), jnp.float32),
        ],
    )(x, embed_t)


def _sc_gather_body(idx_hbm, table_hbm, out_hbm, idx_v, rows_v, sem):
    wid = lax.axis_index("s") * _NC + lax.axis_index("c")
    nt = out_hbm.shape[0]
    per_w = nt // _NW
    base = pl.multiple_of(wid * per_w, _CHUNK)
    for j in range(per_w // _CHUNK):
        off = pl.multiple_of(base + j * _CHUNK, _CHUNK)
        pltpu.sync_copy(idx_hbm.at[pl.ds(off, _CHUNK)], idx_v)
        pltpu.async_copy(table_hbm.at[idx_v], rows_v, sem).wait()
        pltpu.sync_copy(rows_v, out_hbm.at[pl.ds(off, _CHUNK)])


def _sc_gather(idx_flat, embed):
    nt = idx_flat.shape[0]
    mesh = plsc.VectorSubcoreMesh(core_axis_name="c", subcore_axis_name="s")
    fn = pl.kernel(
        _sc_gather_body,
        out_type=jax.ShapeDtypeStruct((nt, D), jnp.float32),
        mesh=mesh,
        scratch_types=[
            pltpu.VMEM((_CHUNK,), jnp.int32),
            pltpu.VMEM((_CHUNK, D), jnp.float32),
            pltpu.SemaphoreType.DMA,
        ],
        compiler_params=pltpu.CompilerParams(use_tc_tiling_on_sc=False),
    )
    return fn(idx_flat, embed)


def kernel(enc, embed):
    B, C, H, W = enc.shape
    x = enc.reshape(-1, D)
    idx_flat, loss = _tc_assign(x, embed.T)
    quantized = _sc_gather(idx_flat, embed)
    quant_out = quantized.reshape(B, C, H, W)
    closest = idx_flat.reshape(B, (C * H * W) // D)
    return (quant_out, loss[0], closest)


# speedup vs baseline: 1.7154x; 1.0486x over previous
import functools

import jax
import jax.numpy as jnp
from jax import lax
from jax.experimental import pallas as pl
from jax.experimental.pallas import tpu as pltpu
from jax.experimental.pallas import tpu_sc as plsc

K = 512
D = 64
TB = 2048

_NC = 2
_NS = 16
_NW = _NC * _NS
_CHUNK = 128
_NBUF = 4


def _tc_body(x_ref, e_ref, idx_ref, loss_ref):
    i = pl.program_id(0)
    x = x_ref[...]
    e = e_ref[...]
    s = lax.dot_general(x, e, (((1,), (0,)), ((), ())),
                        preferred_element_type=jnp.float32)
    q2 = jnp.sum(x * x, axis=1, keepdims=True)
    e2 = jnp.sum(e * e, axis=0)[None, :]
    d2 = (q2 + e2) - 2.0 * s
    idx_ref[...] = jnp.argmin(d2, axis=1).astype(jnp.int32)
    part = jnp.sum(jnp.min(d2, axis=1))

    @pl.when(i == 0)
    def _():
        loss_ref[0] = 0.0

    loss_ref[0] += part

    @pl.when(i == pl.num_programs(0) - 1)
    def _():
        loss_ref[0] = loss_ref[0] * (2.0 / (pl.num_programs(0) * TB * D))


def _tc_assign(x, embed_t):
    nt = x.shape[0]
    grid = nt // TB
    return pl.pallas_call(
        _tc_body,
        grid=(grid,),
        in_specs=[
            pl.BlockSpec((TB, D), lambda i: (i, 0)),
            pl.BlockSpec((D, K), lambda i: (0, 0)),
        ],
        out_specs=[
            pl.BlockSpec((TB,), lambda i: (i,)),
            pl.BlockSpec(memory_space=pltpu.SMEM),
        ],
        out_shape=[
            jax.ShapeDtypeStruct((nt,), jnp.int32),
            jax.ShapeDtypeStruct((1,), jnp.float32),
        ],
    )(x, embed_t)


def _sc_gather_body(idx_hbm, table_hbm, out_hbm, idx_v, rows_v, gsem, osem):
    wid = lax.axis_index("s") * _NC + lax.axis_index("c")
    per_w = idx_hbm.shape[0] // _NW
    nch = per_w // _CHUNK
    base = pl.multiple_of(wid * per_w, _CHUNK)
    gh = [None] * nch
    wh = [None] * nch

    def _start(j):
        b = j % _NBUF
        if j - _NBUF >= 0:
            wh[j - _NBUF].wait()
        off = pl.multiple_of(base + j * _CHUNK, _CHUNK)
        pltpu.sync_copy(idx_hbm.at[pl.ds(off, _CHUNK)], idx_v.at[b])
        gh[j] = pltpu.async_copy(table_hbm.at[idx_v.at[b]], rows_v.at[b], gsem)

    _start(0)
    for j in range(nch):
        if j + 1 < nch:
            _start(j + 1)
        gh[j].wait()
        off = pl.multiple_of(base + j * _CHUNK, _CHUNK)
        wh[j] = pltpu.async_copy(rows_v.at[j % _NBUF],
                                 out_hbm.at[pl.ds(off, _CHUNK)], osem)
    for j in range(max(0, nch - _NBUF), nch):
        wh[j].wait()


def _sc_gather(idx_flat, embed):
    mesh = plsc.VectorSubcoreMesh(core_axis_name="c", subcore_axis_name="s")
    fn = pl.kernel(
        _sc_gather_body,
        out_type=jax.ShapeDtypeStruct((idx_flat.shape[0], D), jnp.float32),
        mesh=mesh,
        scratch_types=[
            pltpu.VMEM((_NBUF, _CHUNK), jnp.int32),
            pltpu.VMEM((_NBUF, _CHUNK, D), jnp.float32),
            pltpu.SemaphoreType.DMA,
            pltpu.SemaphoreType.DMA,
        ],
        compiler_params=pltpu.CompilerParams(use_tc_tiling_on_sc=False),
    )
    return fn(idx_flat, embed)


def kernel(enc, embed):
    B, C, H, W = enc.shape
    x = enc.reshape(-1, D)
    idx_flat, loss = _tc_assign(x, embed.T)
    quant_out = _sc_gather(idx_flat, embed).reshape(B, C, H, W)
    closest = idx_flat.reshape(B, (C * H * W) // D)
    return (quant_out, loss[0], closest)
